# baseline (speedup 1.0000x reference)
import functools

import jax
import jax.numpy as jnp
from jax import lax
from jax.experimental import pallas as pl
from jax.experimental.pallas import tpu as pltpu
from jax.experimental.pallas import tpu_sc as plsc

B, R, D, MAXPOS = 1024, 50, 256, 1000
NLOOK = B * R
NROWS = NLOOK * 4
NC, NS, L = 2, 16, 16
NW = NC * NS
LANES_PER = NROWS // NW
CHUNK = 128
NCHUNKS = LANES_PER // CHUNK
SUBV = CHUNK // L

_mesh = plsc.VectorSubcoreMesh(
    core_axis_name="c", subcore_axis_name="s", num_cores=NC, num_subcores=NS
)


@functools.partial(
    pl.kernel,
    out_type=jax.ShapeDtypeStruct((NROWS, D), jnp.float32),
    mesh=_mesh,
    scratch_types=[
        pltpu.VMEM((LANES_PER + L,), jnp.float32),
        pltpu.VMEM((16,), jnp.float32),
        pltpu.VMEM((NCHUNKS, CHUNK), jnp.int32),
        pltpu.VMEM((2, CHUNK, D), jnp.float32),
        pltpu.SemaphoreType.DMA,
        pltpu.SemaphoreType.DMA,
    ],
)
def _emb_kernel(coord_hbm, scale_hbm, table_hbm, out_hbm,
                coord_v, scale_v, idx_v, rows_v, gsem, ssem):
    wid = lax.axis_index("s") * NC + lax.axis_index("c")
    base = wid * LANES_PER

    pltpu.sync_copy(coord_hbm.at[pl.ds(base, LANES_PER)],
                    coord_v.at[pl.ds(L, LANES_PER)])
    pltpu.sync_copy(scale_hbm, scale_v)

    scale = scale_v[...]
    iota = lax.iota(jnp.int32, L)
    lane4 = iota % 4
    offs = lane4 * MAXPOS
    is_wh = lane4 >= 2

    def compute_chunk(c, _):
        for k in range(SUBV):
            o = c * CHUNK + k * L + L
            raw = coord_v[pl.ds(o, L)]
            other = coord_v[pl.ds(o - 2, L)]
            ia = jnp.clip(raw * scale, 0.0, 999.0).astype(jnp.int32)
            ib = jnp.clip(other * scale, 0.0, 999.0).astype(jnp.int32)
            val = jnp.where(is_wh, jnp.clip(ia - ib, 0, 999), ia) + offs
            idx_v[c, pl.ds(k * L, L)] = val
        return 0

    lax.fori_loop(0, NCHUNKS, compute_chunk, 0)

    def rows_of(c):
        return out_hbm.at[pl.ds(base + c * CHUNK, CHUNK)]

    def step(g, _):
        for b in range(2):
            c = g * 2 + b

            @pl.when(g >= 1)
            def _wait_prev():
                pltpu.make_async_copy(rows_v.at[b], rows_of(c - 2), ssem).wait()

            pltpu.async_copy(table_hbm.at[idx_v.at[c]], rows_v.at[b], gsem).wait()
            pltpu.async_copy(rows_v.at[b], rows_of(c), ssem)
        return 0

    lax.fori_loop(0, NCHUNKS // 2, step, 0)
    for c in (NCHUNKS - 2, NCHUNKS - 1):
        pltpu.make_async_copy(rows_v.at[c % 2], rows_of(c), ssem).wait()


def kernel(bbox, x_table, y_table, w_table, h_table):
    scale = jnp.where(jnp.max(bbox) <= 1.0, 999.0, 1.0)
    scale_vec = jnp.broadcast_to(scale.astype(jnp.float32), (16,))
    coord = bbox.reshape(NROWS)
    table = jnp.concatenate([x_table, y_table, w_table, h_table], axis=0)
    out = _emb_kernel(coord, scale_vec, table)
    return out.reshape(B, R, 4 * D)
